# Initial kernel scaffold; baseline (speedup 1.0000x reference)
import jax
import jax.numpy as jnp
from jax import lax
from jax.experimental import pallas as pl
from jax.experimental.pallas import tpu as pltpu
from jax.experimental.pallas import tpu_sc as plsc

_N = 10000
_E = 320000
_D = 128
_NG = 32

_NC = 2
_NS = 16
_NW = _NC * _NS
_CH = 128
_NCH = 79
_EPW = _NCH * _CH
_EPAD = _NW * _EPW
_NPAD = 10240
_RPT = _NPAD // _NS

_f32 = jnp.float32


def _make_edge_pass(ngrp):
    mesh = plsc.VectorSubcoreMesh(core_axis_name="c", subcore_axis_name="s",
                                  num_cores=_NC, num_subcores=_NS)
    rshape = (_CH, 16) if ngrp == 1 else (_CH, 2, 16)

    def body(a_hbm, b_hbm, src_hbm, dst_hbm, out_hbm,
             acc_sh, sidx, didx, arows, brows, msg, zbuf,
             sem_a, sem_b, sem_i):
        cid = lax.axis_index("c")
        sid = lax.axis_index("s")
        wid = cid * _NS + sid

        zero16 = jnp.zeros((16,), _f32)

        def zb(i, _):
            zbuf[i, 0] = zero16
            zbuf[i, 1] = zero16
            return 0
        lax.fori_loop(0, _RPT, zb, 0)
        pltpu.sync_copy(zbuf, acc_sh.at[pl.ds(sid * _RPT, _RPT)])

        if ngrp == 1:
            lane = lax.broadcasted_iota(jnp.int32, (16,), 0)
            onesv = jnp.where(lane == 0, 1.0, 0.0).astype(_f32)

            def ib(i, _):
                msg[i, 1] = onesv
                return 0
            lax.fori_loop(0, _CH, ib, 0)

        plsc.subcore_barrier()

        def chunk(j, _):
            ci = pltpu.async_copy(src_hbm.at[wid, j], sidx, sem_i)
            cj = pltpu.async_copy(dst_hbm.at[wid, j], didx, sem_i)
            ci.wait()
            cj.wait()
            da = pltpu.async_copy(a_hbm.at[didx], arows, sem_a)
            db = pltpu.async_copy(b_hbm.at[sidx], brows, sem_b)
            da.wait()
            db.wait()

            if ngrp == 1:
                def eb(i, _):
                    msg[i, 0] = jnp.maximum(arows[i] + brows[i], 0.0)
                    return 0
            else:
                def eb(i, _):
                    msg[i, 0] = jnp.maximum(arows[i, 0] + brows[i, 0], 0.0)
                    msg[i, 1] = jnp.maximum(arows[i, 1] + brows[i, 1], 0.0)
                    return 0
            lax.fori_loop(0, _CH, eb, 0)

            pltpu.sync_copy(msg, acc_sh.at[didx], add=True)
            return 0

        lax.fori_loop(0, _NCH, chunk, 0)

        plsc.subcore_barrier()
        pltpu.sync_copy(acc_sh.at[pl.ds(sid * _RPT, _RPT)],
                        out_hbm.at[cid, pl.ds(sid * _RPT, _RPT)])

    return pl.kernel(
        body,
        out_type=jax.ShapeDtypeStruct((_NC, _NPAD, 2, 16), _f32),
        mesh=mesh,
        scratch_types=[
            pltpu.VMEM_SHARED((_NPAD, 2, 16), _f32),
            pltpu.VMEM((_CH,), jnp.int32),
            pltpu.VMEM((_CH,), jnp.int32),
            pltpu.VMEM(rshape, _f32),
            pltpu.VMEM(rshape, _f32),
            pltpu.VMEM((_CH, 2, 16), _f32),
            pltpu.VMEM((_RPT, 2, 16), _f32),
            pltpu.SemaphoreType.DMA,
            pltpu.SemaphoreType.DMA,
            pltpu.SemaphoreType.DMA,
        ],
    )


_edge_pass1 = _make_edge_pass(1)
_edge_pass2 = _make_edge_pass(2)


def _nt(x, w):
    return lax.dot_general(x, w, (((1,), (1,)), ((), ())),
                           preferred_element_type=_f32)


def _tc_tables1(x_ref, w1_ref, b1_ref, a_ref, b_ref):
    x = x_ref[...]
    w = w1_ref[...]
    wa = w[:, :_D]
    wb = w[:, _D:]
    a_ref[pl.ds(0, _N), :] = _nt(x, wa - wb) + b1_ref[...]
    b_ref[pl.ds(0, _N), :] = _nt(x, wb)


def _tc_tables2(p_ref, w2_ref, b2_ref, w1n_ref, b1n_ref,
                a_ref, b_ref, deg_ref):
    s1 = p_ref[0, :, 0, :] + p_ref[1, :, 0, :]
    deg = p_ref[0, :, 1, 0:1] + p_ref[1, :, 1, 0:1]
    x1 = jnp.maximum(_nt(s1, w2_ref[...]) + deg * b2_ref[...], 0.0)
    wn = w1n_ref[...]
    wa = wn[:, :16]
    wb = wn[:, 16:]
    a2 = _nt(x1, wa - wb) + b1n_ref[...]
    b2 = _nt(x1, wb)
    a_ref[:, 0, :] = a2[:, :16]
    a_ref[:, 1, :] = a2[:, 16:]
    b_ref[:, 0, :] = b2[:, :16]
    b_ref[:, 1, :] = b2[:, 16:]
    deg_ref[...] = jnp.broadcast_to(deg, (_NPAD, 8))


def _tc_pool(q_ref, deg_ref, batch_ref, w2_ref, b2_ref,
             gw1_ref, gb1_ref, gw2_ref, gb2_ref,
             cw1_ref, cb1_ref, cw2_ref, cb2_ref, out_ref):
    s2a = q_ref[0, :, 0, :] + q_ref[1, :, 0, :]
    s2b = q_ref[0, :, 1, :] + q_ref[1, :, 1, :]
    w2 = w2_ref[...]
    deg = deg_ref[:, 0:1]
    x2 = jnp.maximum(_nt(s2a, w2[:, :16]) + _nt(s2b, w2[:, 16:])
                     + deg * b2_ref[...], 0.0)
    x2 = x2[pl.ds(0, _N), :]

    g1 = jnp.maximum(_nt(x2, gw1_ref[...]) + gb1_ref[...], 0.0)
    g = _nt(g1, gw2_ref[...]) + gb2_ref[...]

    gid = lax.broadcasted_iota(jnp.int32, (_N, _NG), 1)
    m = batch_ref[...] == gid
    mf = m.astype(_f32)
    gb = jnp.broadcast_to(g, (_N, _NG))
    smax = jnp.max(jnp.where(m, gb, -1e30), axis=0, keepdims=True)
    gmax = jnp.sum(mf * smax, axis=1, keepdims=True)
    e = jnp.exp(g - gmax)
    den = jnp.sum(mf * jnp.broadcast_to(e, (_N, _NG)), axis=0, keepdims=True)
    dn = jnp.sum(mf * den, axis=1, keepdims=True)
    alpha = e / dn
    wts = mf * jnp.broadcast_to(alpha, (_N, _NG))
    pooled = lax.dot_general(wts, x2, (((0,), (0,)), ((), ())),
                             preferred_element_type=_f32)
    h = jnp.maximum(_nt(pooled, cw1_ref[...]) + cb1_ref[...], 0.0)
    out_ref[...] = _nt(h, cw2_ref[...]) + cb2_ref[...]


def kernel(x, edge_index, batch,
           c1_W1, c1_b1, c1_W2, c1_b2,
           c2_W1, c2_b1, c2_W2, c2_b2,
           g_W1, g_b1, g_W2, g_b2,
           cl_W1, cl_b1, cl_W2, cl_b2):
    pad = jnp.full((_EPAD - _E,), _N, jnp.int32)
    srcs = jnp.concatenate([edge_index[0], pad]).reshape(_NW, _NCH, _CH)
    dsts = jnp.concatenate([edge_index[1], pad]).reshape(_NW, _NCH, _CH)

    a1, b1 = pl.pallas_call(
        _tc_tables1,
        out_shape=[jax.ShapeDtypeStruct((_NPAD, 16), _f32),
                   jax.ShapeDtypeStruct((_NPAD, 16), _f32)],
    )(x, c1_W1, c1_b1.reshape(1, 16))

    p1 = _edge_pass1(a1, b1, srcs, dsts)

    a2, b2, deg = pl.pallas_call(
        _tc_tables2,
        out_shape=[jax.ShapeDtypeStruct((_NPAD, 2, 16), _f32),
                   jax.ShapeDtypeStruct((_NPAD, 2, 16), _f32),
                   jax.ShapeDtypeStruct((_NPAD, 8), _f32)],
    )(p1, c1_W2, c1_b2.reshape(1, 16), c2_W1, c2_b1.reshape(1, 32))

    p2 = _edge_pass2(a2, b2, srcs, dsts)

    out = pl.pallas_call(
        _tc_pool,
        out_shape=jax.ShapeDtypeStruct((_NG, 1), _f32),
    )(p2, deg, batch.reshape(_N, 1), c2_W2, c2_b2.reshape(1, 32),
      g_W1, g_b1.reshape(1, 16), g_W2, g_b2.reshape(1, 1),
      cl_W1, cl_b1.reshape(1, 16), cl_W2, cl_b2.reshape(1, 1))
    return out.reshape(_NG)


# speedup vs baseline: 7.0374x; 7.0374x over previous
import jax
import jax.numpy as jnp
from jax import lax
from jax.experimental import pallas as pl
from jax.experimental.pallas import tpu as pltpu
from jax.experimental.pallas import tpu_sc as plsc

_N = 10000
_E = 320000
_D = 128
_NG = 32

_NC = 2
_NS = 16
_NW = _NC * _NS
_CH = 128
_NCH = 79
_EPW = _NCH * _CH
_EPAD = _NW * _EPW
_NPAD = 10240
_RPT = _NPAD // _NS

_f32 = jnp.float32


def _make_edge_pass(ngrp):
    mesh = plsc.VectorSubcoreMesh(core_axis_name="c", subcore_axis_name="s",
                                  num_cores=_NC, num_subcores=_NS)
    rshape = (_CH, 16) if ngrp == 1 else (_CH, 2, 16)

    def body(a_hbm, b_hbm, src_hbm, dst_hbm, out_hbm,
             acc_sh, sidx, didx, arows, brows, msg, zbuf,
             sem_a, sem_b, sem_i):
        cid = lax.axis_index("c")
        sid = lax.axis_index("s")
        wid = cid * _NS + sid

        zero16 = jnp.zeros((16,), _f32)

        def zb(i, _):
            zbuf[i, 0] = zero16
            zbuf[i, 1] = zero16
            return 0
        lax.fori_loop(0, _RPT, zb, 0)
        pltpu.sync_copy(zbuf, acc_sh.at[pl.ds(sid * _RPT, _RPT)])

        if ngrp == 1:
            onesv = jnp.full((16,), 1.0, _f32)

            def ib(i, _):
                msg[i, 1] = onesv
                return 0
            lax.fori_loop(0, _CH, ib, 0)

        plsc.subcore_barrier()

        def chunk(j, _):
            ci = pltpu.async_copy(src_hbm.at[wid, j], sidx, sem_i)
            cj = pltpu.async_copy(dst_hbm.at[wid, j], didx, sem_i)
            ci.wait()
            cj.wait()
            da = pltpu.async_copy(a_hbm.at[didx], arows, sem_a)
            db = pltpu.async_copy(b_hbm.at[sidx], brows, sem_b)
            da.wait()
            db.wait()

            if ngrp == 1:
                def eb(i, _):
                    msg[i, 0] = jnp.maximum(arows[i] + brows[i], 0.0)
                    return 0
            else:
                def eb(i, _):
                    msg[i, 0] = jnp.maximum(arows[i, 0] + brows[i, 0], 0.0)
                    msg[i, 1] = jnp.maximum(arows[i, 1] + brows[i, 1], 0.0)
                    return 0
            lax.fori_loop(0, _CH, eb, 0)

            pltpu.sync_copy(msg, acc_sh.at[didx], add=True)
            return 0

        lax.fori_loop(0, _NCH, chunk, 0)

        plsc.subcore_barrier()
        pltpu.sync_copy(acc_sh.at[pl.ds(sid * _RPT, _RPT)],
                        out_hbm.at[cid, pl.ds(sid * _RPT, _RPT)])

    return pl.kernel(
        body,
        out_type=jax.ShapeDtypeStruct((_NC, _NPAD, 2, 16), _f32),
        mesh=mesh,
        compiler_params=pltpu.CompilerParams(use_tc_tiling_on_sc=False),
        scratch_types=[
            pltpu.VMEM_SHARED((_NPAD, 2, 16), _f32),
            pltpu.VMEM((_CH,), jnp.int32),
            pltpu.VMEM((_CH,), jnp.int32),
            pltpu.VMEM(rshape, _f32),
            pltpu.VMEM(rshape, _f32),
            pltpu.VMEM((_CH, 2, 16), _f32),
            pltpu.VMEM((_RPT, 2, 16), _f32),
            pltpu.SemaphoreType.DMA,
            pltpu.SemaphoreType.DMA,
            pltpu.SemaphoreType.DMA,
        ],
    )


_edge_pass1 = _make_edge_pass(1)
_edge_pass2 = _make_edge_pass(2)


def _nt(x, w):
    return lax.dot_general(x, w, (((1,), (1,)), ((), ())),
                           preferred_element_type=_f32)


def _tc_tables1(x_ref, w1_ref, b1_ref, a_ref, b_ref):
    x = x_ref[...]
    w = w1_ref[...]
    wa = w[:, :_D]
    wb = w[:, _D:]
    a_ref[pl.ds(0, _N), :] = _nt(x, wa - wb) + b1_ref[...]
    b_ref[pl.ds(0, _N), :] = _nt(x, wb)


def _tc_tables2(p_ref, w2_ref, b2_ref, w1n_ref, b1n_ref,
                a_ref, b_ref, deg_ref):
    p0 = p_ref[0]
    p1 = p_ref[1]
    s1 = p0[:, :16] + p1[:, :16]
    deg16 = p0[:, 16:] + p1[:, 16:]
    x1 = jnp.maximum(_nt(s1, w2_ref[...]) + deg16 * b2_ref[...], 0.0)
    wn = w1n_ref[...]
    wa = wn[:, :16]
    wb = wn[:, 16:]
    a_ref[...] = _nt(x1, wa - wb) + b1n_ref[...]
    b_ref[...] = _nt(x1, wb)
    deg_ref[...] = deg16


def _tc_pool(q_ref, deg_ref, batch_ref, w2_ref, b2_ref,
             gw1_ref, gb1_ref, gw2t_ref,
             cw1_ref, cb1_ref, cw2t_ref, cb2_ref, out_ref):
    s2 = q_ref[0] + q_ref[1]
    deg16 = deg_ref[...]
    deg32 = jnp.concatenate([deg16, deg16], axis=1)
    x2 = jnp.maximum(_nt(s2, w2_ref[...]) + deg32 * b2_ref[...], 0.0)
    x2 = lax.slice(x2, (0, 0), (_N, _NG))

    g1 = jnp.maximum(_nt(x2, gw1_ref[...]) + gb1_ref[...], 0.0)
    gmat = _nt(g1, gw2t_ref[...])

    gid = lax.broadcasted_iota(jnp.int32, (_N, _NG), 1)
    m = batch_ref[...] == gid
    mf = m.astype(_f32)
    smax = jnp.max(jnp.where(m, gmat, -1e30), axis=0, keepdims=True)
    smax_b = jnp.broadcast_to(smax, (_N, _NG))
    emat = mf * jnp.exp(jnp.where(m, gmat - smax_b, 0.0))
    den = jnp.sum(emat, axis=0, keepdims=True)
    den = jnp.maximum(den, 1.0)
    alpha = emat / jnp.broadcast_to(den, (_N, _NG))
    pooled = lax.dot_general(alpha, x2, (((0,), (0,)), ((), ())),
                             preferred_element_type=_f32)
    h = jnp.maximum(_nt(pooled, cw1_ref[...]) + cb1_ref[...], 0.0)
    out_ref[...] = _nt(h, cw2t_ref[...]) + cb2_ref[...]


def kernel(x, edge_index, batch,
           c1_W1, c1_b1, c1_W2, c1_b2,
           c2_W1, c2_b1, c2_W2, c2_b2,
           g_W1, g_b1, g_W2, g_b2,
           cl_W1, cl_b1, cl_W2, cl_b2):
    pad = jnp.full((_EPAD - _E,), _N, jnp.int32)
    srcs = jnp.concatenate([edge_index[0], pad]).reshape(_NW, _NCH, _CH)
    dsts = jnp.concatenate([edge_index[1], pad]).reshape(_NW, _NCH, _CH)

    a1, b1 = pl.pallas_call(
        _tc_tables1,
        out_shape=[jax.ShapeDtypeStruct((_NPAD, 16), _f32),
                   jax.ShapeDtypeStruct((_NPAD, 16), _f32)],
    )(x, c1_W1, c1_b1.reshape(1, 16))

    p1 = _edge_pass1(a1, b1, srcs, dsts)

    a2, b2, deg = pl.pallas_call(
        _tc_tables2,
        out_shape=[jax.ShapeDtypeStruct((_NPAD, 32), _f32),
                   jax.ShapeDtypeStruct((_NPAD, 32), _f32),
                   jax.ShapeDtypeStruct((_NPAD, 16), _f32)],
    )(p1.reshape(_NC, _NPAD, 32), c1_W2, c1_b2.reshape(1, 16),
      c2_W1, c2_b1.reshape(1, 32))

    p2 = _edge_pass2(a2.reshape(_NPAD, 2, 16), b2.reshape(_NPAD, 2, 16),
                     srcs, dsts)

    out = pl.pallas_call(
        _tc_pool,
        out_shape=jax.ShapeDtypeStruct((_NG, 16), _f32),
    )(p2.reshape(_NC, _NPAD, 32), deg, batch.reshape(_N, 1),
      c2_W2, c2_b2.reshape(1, 32),
      g_W1, g_b1.reshape(1, 16), jnp.broadcast_to(g_W2, (_NG, 16)),
      cl_W1, cl_b1.reshape(1, 16), jnp.broadcast_to(cl_W2.reshape(1, 16), (16, 16)),
      jnp.broadcast_to(cl_b2.reshape(1, 1), (1, 16)))
    return out[:, 0]


# speedup vs baseline: 9.6125x; 1.3659x over previous
import jax
import jax.numpy as jnp
from jax import lax
from jax.experimental import pallas as pl
from jax.experimental.pallas import tpu as pltpu
from jax.experimental.pallas import tpu_sc as plsc

_N = 10000
_E = 320000
_D = 128
_NG = 32

_NC = 2
_NS = 16
_NW = _NC * _NS
_CH = 128
_NCH = 79
_EPW = _NCH * _CH
_EPAD = _NW * _EPW
_NPAD = 10240
_RPT = _NPAD // _NS

_f32 = jnp.float32


def _make_edge_pass(ngrp):
    mesh = plsc.VectorSubcoreMesh(core_axis_name="c", subcore_axis_name="s",
                                  num_cores=_NC, num_subcores=_NS)
    rshape = (_CH, 16) if ngrp == 1 else (_CH, 2, 16)

    def body(a_hbm, b_hbm, src_hbm, dst_hbm, out_hbm,
             acc_sh, srcall, dstall, arows, brows, msg, zbuf,
             sem_a, sem_b, sem_i, sem_s):
        cid = lax.axis_index("c")
        sid = lax.axis_index("s")
        wid = cid * _NS + sid

        zero16 = jnp.zeros((16,), _f32)

        ci = pltpu.async_copy(src_hbm.at[wid], srcall, sem_i)
        cj = pltpu.async_copy(dst_hbm.at[wid], dstall, sem_i)

        def zb(i, _):
            zbuf[i, 0] = zero16
            zbuf[i, 1] = zero16
            return 0
        lax.fori_loop(0, _RPT, zb, 0)
        pltpu.sync_copy(zbuf, acc_sh.at[pl.ds(sid * _RPT, _RPT)])

        if ngrp == 1:
            onesv = jnp.full((16,), 1.0, _f32)

            def ib(i, _):
                msg[0, i, 1] = onesv
                msg[1, i, 1] = onesv
                return 0
            lax.fori_loop(0, _CH, ib, 0)

        ci.wait()
        cj.wait()
        plsc.subcore_barrier()

        def issue_gather(j, slot):
            pltpu.async_copy(a_hbm.at[dstall.at[j]], arows.at[slot], sem_a)
            pltpu.async_copy(b_hbm.at[srcall.at[j]], brows.at[slot], sem_b)

        issue_gather(0, 0)

        def chunk(j, _):
            slot = lax.rem(j, 2)
            pltpu.make_async_copy(a_hbm.at[dstall.at[j]], arows.at[slot],
                                  sem_a).wait()
            pltpu.make_async_copy(b_hbm.at[srcall.at[j]], brows.at[slot],
                                  sem_b).wait()

            @pl.when(j + 1 < _NCH)
            def _():
                issue_gather(j + 1, lax.rem(j + 1, 2))

            @pl.when(j >= 2)
            def _():
                pltpu.make_async_copy(msg.at[slot], acc_sh.at[dstall.at[j]],
                                      sem_s).wait()

            if ngrp == 1:
                def eb(i):
                    msg[slot, i, 0] = jnp.maximum(
                        arows[slot, i] + brows[slot, i], 0.0)
            else:
                def eb(i):
                    msg[slot, i, 0] = jnp.maximum(
                        arows[slot, i, 0] + brows[slot, i, 0], 0.0)
                    msg[slot, i, 1] = jnp.maximum(
                        arows[slot, i, 1] + brows[slot, i, 1], 0.0)
            plsc.parallel_loop(0, _CH, 1, unroll=8)(eb)

            pltpu.async_copy(msg.at[slot], acc_sh.at[dstall.at[j]], sem_s,
                             add=True)
            return 0

        lax.fori_loop(0, _NCH, chunk, 0)

        pltpu.make_async_copy(msg.at[0], acc_sh.at[dstall.at[0]], sem_s).wait()
        pltpu.make_async_copy(msg.at[1], acc_sh.at[dstall.at[0]], sem_s).wait()

        plsc.subcore_barrier()
        pltpu.sync_copy(acc_sh.at[pl.ds(sid * _RPT, _RPT)],
                        out_hbm.at[cid, pl.ds(sid * _RPT, _RPT)])

    return pl.kernel(
        body,
        out_type=jax.ShapeDtypeStruct((_NC, _NPAD, 2, 16), _f32),
        mesh=mesh,
        compiler_params=pltpu.CompilerParams(use_tc_tiling_on_sc=False),
        scratch_types=[
            pltpu.VMEM_SHARED((_NPAD, 2, 16), _f32),
            pltpu.VMEM((_NCH, _CH), jnp.int32),
            pltpu.VMEM((_NCH, _CH), jnp.int32),
            pltpu.VMEM((2,) + rshape, _f32),
            pltpu.VMEM((2,) + rshape, _f32),
            pltpu.VMEM((2, _CH, 2, 16), _f32),
            pltpu.VMEM((_RPT, 2, 16), _f32),
            pltpu.SemaphoreType.DMA,
            pltpu.SemaphoreType.DMA,
            pltpu.SemaphoreType.DMA,
            pltpu.SemaphoreType.DMA,
        ],
    )


_edge_pass1 = _make_edge_pass(1)
_edge_pass2 = _make_edge_pass(2)


def _nt(x, w):
    return lax.dot_general(x, w, (((1,), (1,)), ((), ())),
                           preferred_element_type=_f32)


def _tc_tables1(x_ref, w1_ref, b1_ref, a_ref, b_ref):
    x = x_ref[...]
    w = w1_ref[...]
    wa = w[:, :_D]
    wb = w[:, _D:]
    a_ref[pl.ds(0, _N), :] = _nt(x, wa - wb) + b1_ref[...]
    b_ref[pl.ds(0, _N), :] = _nt(x, wb)


def _tc_tables2(p_ref, w2_ref, b2_ref, w1n_ref, b1n_ref,
                a_ref, b_ref, deg_ref):
    p0 = p_ref[0]
    p1 = p_ref[1]
    s1 = p0[:, :16] + p1[:, :16]
    deg16 = p0[:, 16:] + p1[:, 16:]
    x1 = jnp.maximum(_nt(s1, w2_ref[...]) + deg16 * b2_ref[...], 0.0)
    wn = w1n_ref[...]
    wa = wn[:, :16]
    wb = wn[:, 16:]
    a_ref[...] = _nt(x1, wa - wb) + b1n_ref[...]
    b_ref[...] = _nt(x1, wb)
    deg_ref[...] = deg16


def _tc_pool(q_ref, deg_ref, batch_ref, w2_ref, b2_ref,
             gw1_ref, gb1_ref, gw2t_ref,
             cw1_ref, cb1_ref, cw2t_ref, cb2_ref, out_ref):
    s2 = q_ref[0] + q_ref[1]
    deg16 = deg_ref[...]
    deg32 = jnp.concatenate([deg16, deg16], axis=1)
    x2 = jnp.maximum(_nt(s2, w2_ref[...]) + deg32 * b2_ref[...], 0.0)
    x2 = lax.slice(x2, (0, 0), (_N, _NG))

    g1 = jnp.maximum(_nt(x2, gw1_ref[...]) + gb1_ref[...], 0.0)
    gmat = _nt(g1, gw2t_ref[...])

    gid = lax.broadcasted_iota(jnp.int32, (_N, _NG), 1)
    m = batch_ref[...] == gid
    mf = m.astype(_f32)
    smax = jnp.max(jnp.where(m, gmat, -1e30), axis=0, keepdims=True)
    smax_b = jnp.broadcast_to(smax, (_N, _NG))
    emat = mf * jnp.exp(jnp.where(m, gmat - smax_b, 0.0))
    den = jnp.sum(emat, axis=0, keepdims=True)
    den = jnp.maximum(den, 1.0)
    alpha = emat / jnp.broadcast_to(den, (_N, _NG))
    pooled = lax.dot_general(alpha, x2, (((0,), (0,)), ((), ())),
                             preferred_element_type=_f32)
    h = jnp.maximum(_nt(pooled, cw1_ref[...]) + cb1_ref[...], 0.0)
    out_ref[...] = _nt(h, cw2t_ref[...]) + cb2_ref[...]


def kernel(x, edge_index, batch,
           c1_W1, c1_b1, c1_W2, c1_b2,
           c2_W1, c2_b1, c2_W2, c2_b2,
           g_W1, g_b1, g_W2, g_b2,
           cl_W1, cl_b1, cl_W2, cl_b2):
    pad = jnp.full((_EPAD - _E,), _N, jnp.int32)
    srcs = jnp.concatenate([edge_index[0], pad]).reshape(_NW, _NCH, _CH)
    dsts = jnp.concatenate([edge_index[1], pad]).reshape(_NW, _NCH, _CH)

    a1, b1 = pl.pallas_call(
        _tc_tables1,
        out_shape=[jax.ShapeDtypeStruct((_NPAD, 16), _f32),
                   jax.ShapeDtypeStruct((_NPAD, 16), _f32)],
    )(x, c1_W1, c1_b1.reshape(1, 16))

    p1 = _edge_pass1(a1, b1, srcs, dsts)

    a2, b2, deg = pl.pallas_call(
        _tc_tables2,
        out_shape=[jax.ShapeDtypeStruct((_NPAD, 32), _f32),
                   jax.ShapeDtypeStruct((_NPAD, 32), _f32),
                   jax.ShapeDtypeStruct((_NPAD, 16), _f32)],
    )(p1.reshape(_NC, _NPAD, 32), c1_W2, c1_b2.reshape(1, 16),
      c2_W1, c2_b1.reshape(1, 32))

    p2 = _edge_pass2(a2.reshape(_NPAD, 2, 16), b2.reshape(_NPAD, 2, 16),
                     srcs, dsts)

    out = pl.pallas_call(
        _tc_pool,
        out_shape=jax.ShapeDtypeStruct((_NG, 16), _f32),
    )(p2.reshape(_NC, _NPAD, 32), deg, batch.reshape(_N, 1),
      c2_W2, c2_b2.reshape(1, 32),
      g_W1, g_b1.reshape(1, 16), jnp.broadcast_to(g_W2, (_NG, 16)),
      cl_W1, cl_b1.reshape(1, 16), jnp.broadcast_to(cl_W2.reshape(1, 16), (16, 16)),
      jnp.broadcast_to(cl_b2.reshape(1, 1), (1, 16)))
    return out[:, 0]


# speedup vs baseline: 20.7359x; 2.1572x over previous
import jax
import jax.numpy as jnp
from jax import lax
from jax.experimental import pallas as pl
from jax.experimental.pallas import tpu as pltpu
from jax.experimental.pallas import tpu_sc as plsc

_N = 10000
_E = 320000
_D = 128
_NG = 32

_NC = 2
_NS = 16
_NW = _NC * _NS
_CH = 128
_NCH = 79
_EPW = _NCH * _CH
_EPAD = _NW * _EPW
_NPAD = 10240
_RPT = _NPAD // _NS

_f32 = jnp.float32


_NBUF = 4


def _make_edge_pass(ngrp):
    mesh = plsc.VectorSubcoreMesh(core_axis_name="c", subcore_axis_name="s",
                                  num_cores=_NC, num_subcores=_NS)
    rw = 16 * ngrp

    def body(a_hbm, b_hbm, src_hbm, dst_hbm, out_hbm,
             acc_sh, srcall, dstall, arows, brows, msg, zbuf,
             sem_a, sem_b, sem_i, sem_s):
        cid = lax.axis_index("c")
        sid = lax.axis_index("s")
        wid = cid * _NS + sid

        zero16 = jnp.zeros((16,), _f32)

        ci = pltpu.async_copy(src_hbm.at[wid], srcall, sem_i)
        cj = pltpu.async_copy(dst_hbm.at[wid], dstall, sem_i)

        def zb(i, _):
            zbuf[i, 0:16] = zero16
            zbuf[i, 16:32] = zero16
            return 0
        lax.fori_loop(0, _RPT, zb, 0)
        pltpu.sync_copy(zbuf, acc_sh.at[pl.ds(sid * _RPT, _RPT)])

        if ngrp == 1:
            onesv = jnp.full((16,), 1.0, _f32)

            def ib(i, _):
                for b in range(_NBUF):
                    msg[b, i, 16:32] = onesv
                return 0
            lax.fori_loop(0, _CH, ib, 0)

        ci.wait()
        cj.wait()
        plsc.subcore_barrier()

        def issue_gather(j):
            slot = lax.rem(j, _NBUF)
            pltpu.async_copy(a_hbm.at[dstall.at[j]], arows.at[slot], sem_a)
            pltpu.async_copy(b_hbm.at[srcall.at[j]], brows.at[slot], sem_b)

        for j0 in range(_NBUF):
            issue_gather(j0)

        def chunk(j, _):
            slot = lax.rem(j, _NBUF)
            pltpu.make_async_copy(a_hbm.at[dstall.at[j]], arows.at[slot],
                                  sem_a).wait()
            pltpu.make_async_copy(b_hbm.at[srcall.at[j]], brows.at[slot],
                                  sem_b).wait()

            @pl.when(j + _NBUF < _NCH)
            def _():
                issue_gather(j + _NBUF)

            @pl.when(j >= _NBUF)
            def _():
                pltpu.make_async_copy(msg.at[slot], acc_sh.at[dstall.at[j]],
                                      sem_s).wait()

            if ngrp == 1:
                def eb(i):
                    msg[slot, i, 0:16] = jnp.maximum(
                        arows[slot, i] + brows[slot, i], 0.0)
            else:
                def eb(i):
                    msg[slot, i, 0:16] = jnp.maximum(
                        arows[slot, i, 0:16] + brows[slot, i, 0:16], 0.0)
                    msg[slot, i, 16:32] = jnp.maximum(
                        arows[slot, i, 16:32] + brows[slot, i, 16:32], 0.0)
            plsc.parallel_loop(0, _CH, 1, unroll=8)(eb)

            pltpu.async_copy(msg.at[slot], acc_sh.at[dstall.at[j]], sem_s,
                             add=True)
            return 0

        lax.fori_loop(0, _NCH, chunk, 0)

        for b in range(_NBUF):
            pltpu.make_async_copy(msg.at[b], acc_sh.at[dstall.at[0]],
                                  sem_s).wait()

        plsc.subcore_barrier()
        pltpu.sync_copy(acc_sh.at[pl.ds(sid * _RPT, _RPT)],
                        out_hbm.at[cid, pl.ds(sid * _RPT, _RPT)])

    return pl.kernel(
        body,
        out_type=jax.ShapeDtypeStruct((_NC, _NPAD, 32), _f32),
        mesh=mesh,
        compiler_params=pltpu.CompilerParams(use_tc_tiling_on_sc=False),
        scratch_types=[
            pltpu.VMEM_SHARED((_NPAD, 32), _f32),
            pltpu.VMEM((_NCH, _CH), jnp.int32),
            pltpu.VMEM((_NCH, _CH), jnp.int32),
            pltpu.VMEM((_NBUF, _CH, rw), _f32),
            pltpu.VMEM((_NBUF, _CH, rw), _f32),
            pltpu.VMEM((_NBUF, _CH, 32), _f32),
            pltpu.VMEM((_RPT, 32), _f32),
            pltpu.SemaphoreType.DMA,
            pltpu.SemaphoreType.DMA,
            pltpu.SemaphoreType.DMA,
            pltpu.SemaphoreType.DMA,
        ],
    )


_edge_pass1 = _make_edge_pass(1)
_edge_pass2 = _make_edge_pass(2)


def _nt(x, w):
    return lax.dot_general(x, w, (((1,), (1,)), ((), ())),
                           preferred_element_type=_f32)


def _tc_tables1(x_ref, w1_ref, b1_ref, a_ref, b_ref):
    x = x_ref[...]
    w = w1_ref[...]
    wa = w[:, :_D]
    wb = w[:, _D:]
    a_ref[pl.ds(0, _N), :] = _nt(x, wa - wb) + b1_ref[...]
    b_ref[pl.ds(0, _N), :] = _nt(x, wb)


def _tc_tables2(p_ref, w2_ref, b2_ref, w1n_ref, b1n_ref,
                a_ref, b_ref, deg_ref):
    p0 = p_ref[0]
    p1 = p_ref[1]
    s1 = p0[:, :16] + p1[:, :16]
    deg16 = p0[:, 16:] + p1[:, 16:]
    x1 = jnp.maximum(_nt(s1, w2_ref[...]) + deg16 * b2_ref[...], 0.0)
    wn = w1n_ref[...]
    wa = wn[:, :16]
    wb = wn[:, 16:]
    a_ref[...] = _nt(x1, wa - wb) + b1n_ref[...]
    b_ref[...] = _nt(x1, wb)
    deg_ref[...] = deg16


def _tc_pool(q_ref, deg_ref, batch_ref, w2_ref, b2_ref,
             gw1_ref, gb1_ref, gw2t_ref,
             cw1_ref, cb1_ref, cw2t_ref, cb2_ref, out_ref):
    s2 = q_ref[0] + q_ref[1]
    deg16 = deg_ref[...]
    deg32 = jnp.concatenate([deg16, deg16], axis=1)
    x2 = jnp.maximum(_nt(s2, w2_ref[...]) + deg32 * b2_ref[...], 0.0)
    x2 = lax.slice(x2, (0, 0), (_N, _NG))

    g1 = jnp.maximum(_nt(x2, gw1_ref[...]) + gb1_ref[...], 0.0)
    gmat = _nt(g1, gw2t_ref[...])

    gid = lax.broadcasted_iota(jnp.int32, (_N, _NG), 1)
    m = batch_ref[...] == gid
    mf = m.astype(_f32)
    smax = jnp.max(jnp.where(m, gmat, -1e30), axis=0, keepdims=True)
    smax_b = jnp.broadcast_to(smax, (_N, _NG))
    emat = mf * jnp.exp(jnp.where(m, gmat - smax_b, 0.0))
    den = jnp.sum(emat, axis=0, keepdims=True)
    den = jnp.maximum(den, 1.0)
    alpha = emat / jnp.broadcast_to(den, (_N, _NG))
    pooled = lax.dot_general(alpha, x2, (((0,), (0,)), ((), ())),
                             preferred_element_type=_f32)
    h = jnp.maximum(_nt(pooled, cw1_ref[...]) + cb1_ref[...], 0.0)
    out_ref[...] = _nt(h, cw2t_ref[...]) + cb2_ref[...]


def kernel(x, edge_index, batch,
           c1_W1, c1_b1, c1_W2, c1_b2,
           c2_W1, c2_b1, c2_W2, c2_b2,
           g_W1, g_b1, g_W2, g_b2,
           cl_W1, cl_b1, cl_W2, cl_b2):
    pad = jnp.full((_EPAD - _E,), _N, jnp.int32)
    srcs = jnp.concatenate([edge_index[0], pad]).reshape(_NW, _NCH, _CH)
    dsts = jnp.concatenate([edge_index[1], pad]).reshape(_NW, _NCH, _CH)

    a1, b1 = pl.pallas_call(
        _tc_tables1,
        out_shape=[jax.ShapeDtypeStruct((_NPAD, 16), _f32),
                   jax.ShapeDtypeStruct((_NPAD, 16), _f32)],
    )(x, c1_W1, c1_b1.reshape(1, 16))

    p1 = _edge_pass1(a1, b1, srcs, dsts)

    a2, b2, deg = pl.pallas_call(
        _tc_tables2,
        out_shape=[jax.ShapeDtypeStruct((_NPAD, 32), _f32),
                   jax.ShapeDtypeStruct((_NPAD, 32), _f32),
                   jax.ShapeDtypeStruct((_NPAD, 16), _f32)],
    )(p1, c1_W2, c1_b2.reshape(1, 16), c2_W1, c2_b1.reshape(1, 32))

    p2 = _edge_pass2(a2, b2, srcs, dsts)

    out = pl.pallas_call(
        _tc_pool,
        out_shape=jax.ShapeDtypeStruct((_NG, 16), _f32),
    )(p2, deg, batch.reshape(_N, 1),
      c2_W2, c2_b2.reshape(1, 32),
      g_W1, g_b1.reshape(1, 16), jnp.broadcast_to(g_W2, (_NG, 16)),
      cl_W1, cl_b1.reshape(1, 16), jnp.broadcast_to(cl_W2.reshape(1, 16), (16, 16)),
      jnp.broadcast_to(cl_b2.reshape(1, 1), (1, 16)))
    return out[:, 0]


# speedup vs baseline: 21.9118x; 1.0567x over previous
import jax
import jax.numpy as jnp
from jax import lax
from jax.experimental import pallas as pl
from jax.experimental.pallas import tpu as pltpu
from jax.experimental.pallas import tpu_sc as plsc

_N = 10000
_E = 320000
_D = 128
_NG = 32

_NC = 2
_NS = 16
_NW = _NC * _NS
_CH = 128
_TCH = 2528
_EPAD = _TCH * _CH
_NCH0 = 95
_NCH1 = _TCH // _NS - _NCH0
_NPAD = 10240
_RPT = _NPAD // _NS
_ZR = 128

_f32 = jnp.float32


_NBUF = 6


def _make_edge_pass(ngrp):
    mesh = plsc.VectorSubcoreMesh(core_axis_name="c", subcore_axis_name="s",
                                  num_cores=_NC, num_subcores=_NS)
    rw = 16 * ngrp

    def body(a_hbm, b_hbm, src_hbm, dst_hbm, out_hbm,
             acc_sh, srcall, dstall, arows, brows, msg, zbuf,
             sem_a, sem_b, sem_i, sem_s):
        cid = lax.axis_index("c")
        sid = lax.axis_index("s")
        nch = jnp.where(cid == 0, _NCH0, _NCH1)

        @pl.when(cid == 0)
        def _():
            s0 = pl.ds(sid * _NCH0, _NCH0)
            pltpu.async_copy(src_hbm.at[s0], srcall.at[pl.ds(0, _NCH0)],
                             sem_i)
            pltpu.async_copy(dst_hbm.at[s0], dstall.at[pl.ds(0, _NCH0)],
                             sem_i)

        @pl.when(cid == 1)
        def _():
            s1 = pl.ds(_NS * _NCH0 + sid * _NCH1, _NCH1)
            pltpu.async_copy(src_hbm.at[s1], srcall.at[pl.ds(0, _NCH1)],
                             sem_i)
            pltpu.async_copy(dst_hbm.at[s1], dstall.at[pl.ds(0, _NCH1)],
                             sem_i)

        zero16 = jnp.zeros((16,), _f32)

        def zb(i, _):
            zbuf[i, 0:16] = zero16
            zbuf[i, 16:32] = zero16
            return 0
        lax.fori_loop(0, _ZR, zb, 0)
        for k in range(_RPT // _ZR):
            pltpu.sync_copy(zbuf, acc_sh.at[pl.ds(sid * _RPT + k * _ZR, _ZR)])

        if ngrp == 1:
            onesv = jnp.full((16,), 1.0, _f32)

            def ib(i, _):
                for b in range(_NBUF):
                    msg[b, i, 16:32] = onesv
                return 0
            lax.fori_loop(0, _CH, ib, 0)

        @pl.when(cid == 0)
        def _():
            for _u in range(2):
                pltpu.make_async_copy(src_hbm.at[pl.ds(0, _NCH0)],
                                      srcall.at[pl.ds(0, _NCH0)],
                                      sem_i).wait()

        @pl.when(cid == 1)
        def _():
            for _u in range(2):
                pltpu.make_async_copy(src_hbm.at[pl.ds(0, _NCH1)],
                                      srcall.at[pl.ds(0, _NCH1)],
                                      sem_i).wait()

        plsc.subcore_barrier()

        def issue_gather(j):
            slot = lax.rem(j, _NBUF)
            pltpu.async_copy(a_hbm.at[dstall.at[j]], arows.at[slot], sem_a)
            pltpu.async_copy(b_hbm.at[srcall.at[j]], brows.at[slot], sem_b)

        for j0 in range(_NBUF):
            issue_gather(j0)

        def chunk(j, _):
            slot = lax.rem(j, _NBUF)
            pltpu.make_async_copy(a_hbm.at[dstall.at[j]], arows.at[slot],
                                  sem_a).wait()
            pltpu.make_async_copy(b_hbm.at[srcall.at[j]], brows.at[slot],
                                  sem_b).wait()

            @pl.when(j >= _NBUF)
            def _():
                pltpu.make_async_copy(msg.at[slot], acc_sh.at[dstall.at[j]],
                                      sem_s).wait()

            if ngrp == 1:
                def eb(i):
                    msg[slot, i, 0:16] = jnp.maximum(
                        arows[slot, i] + brows[slot, i], 0.0)
            else:
                def eb(i):
                    msg[slot, i, 0:16] = jnp.maximum(
                        arows[slot, i, 0:16] + brows[slot, i, 0:16], 0.0)
                    msg[slot, i, 16:32] = jnp.maximum(
                        arows[slot, i, 16:32] + brows[slot, i, 16:32], 0.0)
            plsc.parallel_loop(0, _CH, 1, unroll=8)(eb)

            @pl.when(j + _NBUF < nch)
            def _():
                issue_gather(j + _NBUF)

            pltpu.async_copy(msg.at[slot], acc_sh.at[dstall.at[j]], sem_s,
                             add=True)
            return 0

        lax.fori_loop(0, nch, chunk, 0)

        for b in range(_NBUF):
            pltpu.make_async_copy(msg.at[b], acc_sh.at[dstall.at[0]],
                                  sem_s).wait()

        plsc.subcore_barrier()
        pltpu.sync_copy(acc_sh.at[pl.ds(sid * _RPT, _RPT)],
                        out_hbm.at[cid, pl.ds(sid * _RPT, _RPT)])

    return pl.kernel(
        body,
        out_type=jax.ShapeDtypeStruct((_NC, _NPAD, 32), _f32),
        mesh=mesh,
        compiler_params=pltpu.CompilerParams(use_tc_tiling_on_sc=False),
        scratch_types=[
            pltpu.VMEM_SHARED((_NPAD, 32), _f32),
            pltpu.VMEM((_NCH0, _CH), jnp.int32),
            pltpu.VMEM((_NCH0, _CH), jnp.int32),
            pltpu.VMEM((_NBUF, _CH, rw), _f32),
            pltpu.VMEM((_NBUF, _CH, rw), _f32),
            pltpu.VMEM((_NBUF, _CH, 32), _f32),
            pltpu.VMEM((_ZR, 32), _f32),
            pltpu.SemaphoreType.DMA,
            pltpu.SemaphoreType.DMA,
            pltpu.SemaphoreType.DMA,
            pltpu.SemaphoreType.DMA,
        ],
    )


_edge_pass1 = _make_edge_pass(1)
_edge_pass2 = _make_edge_pass(2)


def _nt(x, w):
    return lax.dot_general(x, w, (((1,), (1,)), ((), ())),
                           preferred_element_type=_f32)


def _tc_tables1(x_ref, w1_ref, b1_ref, a_ref, b_ref):
    x = x_ref[...]
    w = w1_ref[...]
    wa = w[:, :_D]
    wb = w[:, _D:]
    a_ref[pl.ds(0, _N), :] = _nt(x, wa - wb) + b1_ref[...]
    b_ref[pl.ds(0, _N), :] = _nt(x, wb)


def _tc_tables2(p_ref, w2_ref, b2_ref, w1n_ref, b1n_ref,
                a_ref, b_ref, deg_ref):
    p0 = p_ref[0]
    p1 = p_ref[1]
    s1 = p0[:, :16] + p1[:, :16]
    deg16 = p0[:, 16:] + p1[:, 16:]
    x1 = jnp.maximum(_nt(s1, w2_ref[...]) + deg16 * b2_ref[...], 0.0)
    wn = w1n_ref[...]
    wa = wn[:, :16]
    wb = wn[:, 16:]
    a_ref[...] = _nt(x1, wa - wb) + b1n_ref[...]
    b_ref[...] = _nt(x1, wb)
    deg_ref[...] = deg16


def _tc_pool(q_ref, deg_ref, batch_ref, w2_ref, b2_ref,
             gw1_ref, gb1_ref, gw2t_ref,
             cw1_ref, cb1_ref, cw2t_ref, cb2_ref, out_ref):
    s2 = q_ref[0] + q_ref[1]
    deg16 = deg_ref[...]
    deg32 = jnp.concatenate([deg16, deg16], axis=1)
    x2 = jnp.maximum(_nt(s2, w2_ref[...]) + deg32 * b2_ref[...], 0.0)
    x2 = lax.slice(x2, (0, 0), (_N, _NG))

    g1 = jnp.maximum(_nt(x2, gw1_ref[...]) + gb1_ref[...], 0.0)
    gmat = _nt(g1, gw2t_ref[...])

    gid = lax.broadcasted_iota(jnp.int32, (_N, _NG), 1)
    m = batch_ref[...] == gid
    mf = m.astype(_f32)
    smax = jnp.max(jnp.where(m, gmat, -1e30), axis=0, keepdims=True)
    smax_b = jnp.broadcast_to(smax, (_N, _NG))
    emat = mf * jnp.exp(jnp.where(m, gmat - smax_b, 0.0))
    den = jnp.sum(emat, axis=0, keepdims=True)
    den = jnp.maximum(den, 1.0)
    alpha = emat / jnp.broadcast_to(den, (_N, _NG))
    pooled = lax.dot_general(alpha, x2, (((0,), (0,)), ((), ())),
                             preferred_element_type=_f32)
    h = jnp.maximum(_nt(pooled, cw1_ref[...]) + cb1_ref[...], 0.0)
    out_ref[...] = _nt(h, cw2t_ref[...]) + cb2_ref[...]


def kernel(x, edge_index, batch,
           c1_W1, c1_b1, c1_W2, c1_b2,
           c2_W1, c2_b1, c2_W2, c2_b2,
           g_W1, g_b1, g_W2, g_b2,
           cl_W1, cl_b1, cl_W2, cl_b2):
    pad = jnp.full((_EPAD - _E,), _N, jnp.int32)
    srcs = jnp.concatenate([edge_index[0], pad]).reshape(_TCH, _CH)
    dsts = jnp.concatenate([edge_index[1], pad]).reshape(_TCH, _CH)

    a1, b1 = pl.pallas_call(
        _tc_tables1,
        out_shape=[jax.ShapeDtypeStruct((_NPAD, 16), _f32),
                   jax.ShapeDtypeStruct((_NPAD, 16), _f32)],
    )(x, c1_W1, c1_b1.reshape(1, 16))

    p1 = _edge_pass1(a1, b1, srcs, dsts)

    a2, b2, deg = pl.pallas_call(
        _tc_tables2,
        out_shape=[jax.ShapeDtypeStruct((_NPAD, 32), _f32),
                   jax.ShapeDtypeStruct((_NPAD, 32), _f32),
                   jax.ShapeDtypeStruct((_NPAD, 16), _f32)],
    )(p1, c1_W2, c1_b2.reshape(1, 16), c2_W1, c2_b1.reshape(1, 32))

    p2 = _edge_pass2(a2, b2, srcs, dsts)

    out = pl.pallas_call(
        _tc_pool,
        out_shape=jax.ShapeDtypeStruct((_NG, 16), _f32),
    )(p2, deg, batch.reshape(_N, 1),
      c2_W2, c2_b2.reshape(1, 32),
      g_W1, g_b1.reshape(1, 16), jnp.broadcast_to(g_W2, (_NG, 16)),
      cl_W1, cl_b1.reshape(1, 16), jnp.broadcast_to(cl_W2.reshape(1, 16), (16, 16)),
      jnp.broadcast_to(cl_b2.reshape(1, 1), (1, 16)))
    return out[:, 0]


# speedup vs baseline: 22.4186x; 1.0231x over previous
import jax
import jax.numpy as jnp
from jax import lax
from jax.experimental import pallas as pl
from jax.experimental.pallas import tpu as pltpu
from jax.experimental.pallas import tpu_sc as plsc

_N = 10000
_E = 320000
_D = 128
_NG = 32

_NC = 2
_NS = 16
_NW = _NC * _NS
_CH = 128
_TCH = 2528
_EPAD = _TCH * _CH
_NCH0 = 106
_NCH1 = _TCH // _NS - _NCH0
_NPAD = 10240
_RPT = _NPAD // _NS
_ZR = 128

_f32 = jnp.float32


_NBUF = 6


def _make_edge_pass(ngrp):
    mesh = plsc.VectorSubcoreMesh(core_axis_name="c", subcore_axis_name="s",
                                  num_cores=_NC, num_subcores=_NS)
    rw = 16 * ngrp

    def body(a_hbm, b_hbm, src_hbm, dst_hbm, out_hbm,
             acc_sh, srcall, dstall, arows, brows, msg, zbuf,
             sem_a, sem_b, sem_i, sem_s):
        cid = lax.axis_index("c")
        sid = lax.axis_index("s")
        nch = jnp.where(cid == 0, _NCH0, _NCH1)

        @pl.when(cid == 0)
        def _():
            s0 = pl.ds(sid * _NCH0, _NCH0)
            pltpu.async_copy(src_hbm.at[s0], srcall.at[pl.ds(0, _NCH0)],
                             sem_i)
            pltpu.async_copy(dst_hbm.at[s0], dstall.at[pl.ds(0, _NCH0)],
                             sem_i)

        @pl.when(cid == 1)
        def _():
            s1 = pl.ds(_NS * _NCH0 + sid * _NCH1, _NCH1)
            pltpu.async_copy(src_hbm.at[s1], srcall.at[pl.ds(0, _NCH1)],
                             sem_i)
            pltpu.async_copy(dst_hbm.at[s1], dstall.at[pl.ds(0, _NCH1)],
                             sem_i)

        zero16 = jnp.zeros((16,), _f32)

        def zb(i, _):
            zbuf[i, 0:16] = zero16
            zbuf[i, 16:32] = zero16
            return 0
        lax.fori_loop(0, _ZR, zb, 0)
        for k in range(_RPT // _ZR):
            pltpu.sync_copy(zbuf, acc_sh.at[pl.ds(sid * _RPT + k * _ZR, _ZR)])

        if ngrp == 1:
            onesv = jnp.full((16,), 1.0, _f32)

            def ib(i, _):
                for b in range(_NBUF):
                    msg[b, i, 16:32] = onesv
                return 0
            lax.fori_loop(0, _CH, ib, 0)

        @pl.when(cid == 0)
        def _():
            for _u in range(2):
                pltpu.make_async_copy(src_hbm.at[pl.ds(0, _NCH0)],
                                      srcall.at[pl.ds(0, _NCH0)],
                                      sem_i).wait()

        @pl.when(cid == 1)
        def _():
            for _u in range(2):
                pltpu.make_async_copy(src_hbm.at[pl.ds(0, _NCH1)],
                                      srcall.at[pl.ds(0, _NCH1)],
                                      sem_i).wait()

        plsc.subcore_barrier()

        def issue_gather(j):
            slot = lax.rem(j, _NBUF)
            pltpu.async_copy(a_hbm.at[dstall.at[j]], arows.at[slot], sem_a)
            pltpu.async_copy(b_hbm.at[srcall.at[j]], brows.at[slot], sem_b)

        for j0 in range(_NBUF):
            issue_gather(j0)

        def chunk(j, _):
            slot = lax.rem(j, _NBUF)
            pltpu.make_async_copy(a_hbm.at[dstall.at[j]], arows.at[slot],
                                  sem_a).wait()
            pltpu.make_async_copy(b_hbm.at[srcall.at[j]], brows.at[slot],
                                  sem_b).wait()

            @pl.when(j >= _NBUF)
            def _():
                pltpu.make_async_copy(msg.at[slot], acc_sh.at[dstall.at[j]],
                                      sem_s).wait()

            if ngrp == 1:
                def eb(i):
                    msg[slot, i, 0:16] = jnp.maximum(
                        arows[slot, i] + brows[slot, i], 0.0)
            else:
                def eb(i):
                    msg[slot, i, 0:16] = jnp.maximum(
                        arows[slot, i, 0:16] + brows[slot, i, 0:16], 0.0)
                    msg[slot, i, 16:32] = jnp.maximum(
                        arows[slot, i, 16:32] + brows[slot, i, 16:32], 0.0)
            plsc.parallel_loop(0, _CH, 1, unroll=8)(eb)

            @pl.when(j + _NBUF < nch)
            def _():
                issue_gather(j + _NBUF)

            pltpu.async_copy(msg.at[slot], acc_sh.at[dstall.at[j]], sem_s,
                             add=True)
            return 0

        lax.fori_loop(0, nch, chunk, 0)

        for b in range(_NBUF):
            pltpu.make_async_copy(msg.at[b], acc_sh.at[dstall.at[0]],
                                  sem_s).wait()

        plsc.subcore_barrier()
        pltpu.sync_copy(acc_sh.at[pl.ds(sid * _RPT, _RPT)],
                        out_hbm.at[cid, pl.ds(sid * _RPT, _RPT)])

    return pl.kernel(
        body,
        out_type=jax.ShapeDtypeStruct((_NC, _NPAD, 32), _f32),
        mesh=mesh,
        compiler_params=pltpu.CompilerParams(use_tc_tiling_on_sc=False),
        scratch_types=[
            pltpu.VMEM_SHARED((_NPAD, 32), _f32),
            pltpu.VMEM((_NCH0, _CH), jnp.int32),
            pltpu.VMEM((_NCH0, _CH), jnp.int32),
            pltpu.VMEM((_NBUF, _CH, rw), _f32),
            pltpu.VMEM((_NBUF, _CH, rw), _f32),
            pltpu.VMEM((_NBUF, _CH, 32), _f32),
            pltpu.VMEM((_ZR, 32), _f32),
            pltpu.SemaphoreType.DMA,
            pltpu.SemaphoreType.DMA,
            pltpu.SemaphoreType.DMA,
            pltpu.SemaphoreType.DMA,
        ],
    )


_edge_pass1 = _make_edge_pass(1)
_edge_pass2 = _make_edge_pass(2)


def _nt(x, w):
    return lax.dot_general(x, w, (((1,), (1,)), ((), ())),
                           preferred_element_type=_f32)


def _tc_tables1(x_ref, w1_ref, b1_ref, a_ref, b_ref):
    x = x_ref[...]
    w = w1_ref[...]
    wa = w[:, :_D]
    wb = w[:, _D:]
    a_ref[pl.ds(0, _N), :] = _nt(x, wa - wb) + b1_ref[...]
    b_ref[pl.ds(0, _N), :] = _nt(x, wb)


def _tc_tables2(p_ref, w2_ref, b2_ref, w1n_ref, b1n_ref,
                a_ref, b_ref, deg_ref):
    p0 = p_ref[0]
    p1 = p_ref[1]
    s1 = p0[:, :16] + p1[:, :16]
    deg16 = p0[:, 16:] + p1[:, 16:]
    x1 = jnp.maximum(_nt(s1, w2_ref[...]) + deg16 * b2_ref[...], 0.0)
    wn = w1n_ref[...]
    wa = wn[:, :16]
    wb = wn[:, 16:]
    a_ref[...] = _nt(x1, wa - wb) + b1n_ref[...]
    b_ref[...] = _nt(x1, wb)
    deg_ref[...] = deg16


def _tc_pool(q_ref, deg_ref, batch_ref, w2_ref, b2_ref,
             gw1_ref, gb1_ref, gw2t_ref,
             cw1_ref, cb1_ref, cw2t_ref, cb2_ref, out_ref):
    s2 = q_ref[0] + q_ref[1]
    deg16 = deg_ref[...]
    deg32 = jnp.concatenate([deg16, deg16], axis=1)
    x2 = jnp.maximum(_nt(s2, w2_ref[...]) + deg32 * b2_ref[...], 0.0)
    x2 = lax.slice(x2, (0, 0), (_N, _NG))

    g1 = jnp.maximum(_nt(x2, gw1_ref[...]) + gb1_ref[...], 0.0)
    gmat = _nt(g1, gw2t_ref[...])

    gid = lax.broadcasted_iota(jnp.int32, (_N, _NG), 1)
    m = batch_ref[...] == gid
    mf = m.astype(_f32)
    smax = jnp.max(jnp.where(m, gmat, -1e30), axis=0, keepdims=True)
    smax_b = jnp.broadcast_to(smax, (_N, _NG))
    emat = mf * jnp.exp(jnp.where(m, gmat - smax_b, 0.0))
    den = jnp.sum(emat, axis=0, keepdims=True)
    den = jnp.maximum(den, 1.0)
    alpha = emat / jnp.broadcast_to(den, (_N, _NG))
    pooled = lax.dot_general(alpha, x2, (((0,), (0,)), ((), ())),
                             preferred_element_type=_f32)
    h = jnp.maximum(_nt(pooled, cw1_ref[...]) + cb1_ref[...], 0.0)
    out_ref[...] = _nt(h, cw2t_ref[...]) + cb2_ref[...]


def kernel(x, edge_index, batch,
           c1_W1, c1_b1, c1_W2, c1_b2,
           c2_W1, c2_b1, c2_W2, c2_b2,
           g_W1, g_b1, g_W2, g_b2,
           cl_W1, cl_b1, cl_W2, cl_b2):
    pad = jnp.full((_EPAD - _E,), _N, jnp.int32)
    srcs = jnp.concatenate([edge_index[0], pad]).reshape(_TCH, _CH)
    dsts = jnp.concatenate([edge_index[1], pad]).reshape(_TCH, _CH)

    a1, b1 = pl.pallas_call(
        _tc_tables1,
        out_shape=[jax.ShapeDtypeStruct((_NPAD, 16), _f32),
                   jax.ShapeDtypeStruct((_NPAD, 16), _f32)],
    )(x, c1_W1, c1_b1.reshape(1, 16))

    p1 = _edge_pass1(a1, b1, srcs, dsts)

    a2, b2, deg = pl.pallas_call(
        _tc_tables2,
        out_shape=[jax.ShapeDtypeStruct((_NPAD, 32), _f32),
                   jax.ShapeDtypeStruct((_NPAD, 32), _f32),
                   jax.ShapeDtypeStruct((_NPAD, 16), _f32)],
    )(p1, c1_W2, c1_b2.reshape(1, 16), c2_W1, c2_b1.reshape(1, 32))

    p2 = _edge_pass2(a2, b2, srcs, dsts)

    out = pl.pallas_call(
        _tc_pool,
        out_shape=jax.ShapeDtypeStruct((_NG, 16), _f32),
    )(p2, deg, batch.reshape(_N, 1),
      c2_W2, c2_b2.reshape(1, 32),
      g_W1, g_b1.reshape(1, 16), jnp.broadcast_to(g_W2, (_NG, 16)),
      cl_W1, cl_b1.reshape(1, 16), jnp.broadcast_to(cl_W2.reshape(1, 16), (16, 16)),
      jnp.broadcast_to(cl_b2.reshape(1, 1), (1, 16)))
    return out[:, 0]
